# Initial kernel scaffold; baseline (speedup 1.0000x reference)
import jax
import jax.numpy as jnp
from jax.experimental import pallas as pl


def kernel(x, sw, sb, b0_c1w, b0_c1b, b0_c2w, b0_c2b, b0_c3w, b0_c3b, b0_dw, b0_db, b1_c1w, b1_c1b, b1_c2w, b1_c2b, b1_c3w, b1_c3b, b2_c1w, b2_c1b, b2_c2w, b2_c2b, b2_c3w, b2_c3b, b3_c1w, b3_c1b, b3_c2w, b3_c2b, b3_c3w, b3_c3b, b3_dw, b3_db, b4_c1w, b4_c1b, b4_c2w, b4_c2b, b4_c3w, b4_c3b, b5_c1w, b5_c1b, b5_c2w, b5_c2b, b5_c3w, b5_c3b, b6_c1w, b6_c1b, b6_c2w, b6_c2b, b6_c3w, b6_c3b, b7_c1w, b7_c1b, b7_c2w, b7_c2b, b7_c3w, b7_c3b, b8_c1w, b8_c1b, b8_c2w, b8_c2b, b8_c3w, b8_c3b, b9_c1w, b9_c1b, b9_c2w, b9_c2b, b9_c3w, b9_c3b, b10_c1w, b10_c1b, b10_c2w, b10_c2b, b10_c3w, b10_c3b, b11_c1w, b11_c1b, b11_c2w, b11_c2b, b11_c3w, b11_c3b, b11_dw, b11_db, b12_c1w, b12_c1b, b12_c2w, b12_c2b, b12_c3w, b12_c3b, b13_c1w, b13_c1b, b13_c2w, b13_c2b, b13_c3w, b13_c3b, b14_c1w, b14_c1b, b14_c2w, b14_c2b, b14_c3w, b14_c3b, b15_c1w, b15_c1b, b15_c2w, b15_c2b, b15_c3w, b15_c3b, b16_c1w, b16_c1b, b16_c2w, b16_c2b, b16_c3w, b16_c3b, b17_c1w, b17_c1b, b17_c2w, b17_c2b, b17_c3w, b17_c3b, b18_c1w, b18_c1b, b18_c2w, b18_c2b, b18_c3w, b18_c3b, b19_c1w, b19_c1b, b19_c2w, b19_c2b, b19_c3w, b19_c3b, b20_c1w, b20_c1b, b20_c2w, b20_c2b, b20_c3w, b20_c3b, b21_c1w, b21_c1b, b21_c2w, b21_c2b, b21_c3w, b21_c3b, b22_c1w, b22_c1b, b22_c2w, b22_c2b, b22_c3w, b22_c3b, b23_c1w, b23_c1b, b23_c2w, b23_c2b, b23_c3w, b23_c3b, b24_c1w, b24_c1b, b24_c2w, b24_c2b, b24_c3w, b24_c3b, b25_c1w, b25_c1b, b25_c2w, b25_c2b, b25_c3w, b25_c3b, b26_c1w, b26_c1b, b26_c2w, b26_c2b, b26_c3w, b26_c3b, b27_c1w, b27_c1b, b27_c2w, b27_c2b, b27_c3w, b27_c3b, b28_c1w, b28_c1b, b28_c2w, b28_c2b, b28_c3w, b28_c3b, b29_c1w, b29_c1b, b29_c2w, b29_c2b, b29_c3w, b29_c3b, b30_c1w, b30_c1b, b30_c2w, b30_c2b, b30_c3w, b30_c3b, b31_c1w, b31_c1b, b31_c2w, b31_c2b, b31_c3w, b31_c3b, b32_c1w, b32_c1b, b32_c2w, b32_c2b, b32_c3w, b32_c3b, b33_c1w, b33_c1b, b33_c2w, b33_c2b, b33_c3w, b33_c3b, b34_c1w, b34_c1b, b34_c2w, b34_c2b, b34_c3w, b34_c3b, b35_c1w, b35_c1b, b35_c2w, b35_c2b, b35_c3w, b35_c3b, b36_c1w, b36_c1b, b36_c2w, b36_c2b, b36_c3w, b36_c3b, b37_c1w, b37_c1b, b37_c2w, b37_c2b, b37_c3w, b37_c3b, b38_c1w, b38_c1b, b38_c2w, b38_c2b, b38_c3w, b38_c3b, b39_c1w, b39_c1b, b39_c2w, b39_c2b, b39_c3w, b39_c3b, b40_c1w, b40_c1b, b40_c2w, b40_c2b, b40_c3w, b40_c3b, b41_c1w, b41_c1b, b41_c2w, b41_c2b, b41_c3w, b41_c3b, b42_c1w, b42_c1b, b42_c2w, b42_c2b, b42_c3w, b42_c3b, b43_c1w, b43_c1b, b43_c2w, b43_c2b, b43_c3w, b43_c3b, b44_c1w, b44_c1b, b44_c2w, b44_c2b, b44_c3w, b44_c3b, b45_c1w, b45_c1b, b45_c2w, b45_c2b, b45_c3w, b45_c3b, b46_c1w, b46_c1b, b46_c2w, b46_c2b, b46_c3w, b46_c3b, b47_c1w, b47_c1b, b47_c2w, b47_c2b, b47_c3w, b47_c3b, b47_dw, b47_db, b48_c1w, b48_c1b, b48_c2w, b48_c2b, b48_c3w, b48_c3b, b49_c1w, b49_c1b, b49_c2w, b49_c2b, b49_c3w, b49_c3b, e0_w, e0_b, e1_w, e1_b, e2_w, e2_b, e3_w, e3_b, e4_w, e4_b, e5_w, e5_b, e6_w, e6_b, e7_w, e7_b, e8_w, e8_b, e9_w, e9_b, e10_w, e10_b, e11_w, e11_b, e12_w, e12_b, e13_w, e13_b, e14_w, e14_b, e15_w, e15_b, e16_w, e16_b, e17_w, e17_b, e18_w, e18_b, e19_w, e19_b, e20_w, e20_b, e21_w, e21_b, e22_w, e22_b, e23_w, e23_b, e24_w, e24_b, e25_w, e25_b, e26_w, e26_b, e27_w, e27_b, e28_w, e28_b, e29_w, e29_b, e30_w, e30_b, e31_w, e31_b, e32_w, e32_b, e33_w, e33_b, e34_w, e34_b, e35_w, e35_b, e36_w, e36_b, e37_w, e37_b, e38_w, e38_b, e39_w, e39_b, e40_w, e40_b, e41_w, e41_b, e42_w, e42_b, e43_w, e43_b, e44_w, e44_b, e45_w, e45_b, e46_w, e46_b, fcw, fcb):
    raise NotImplementedError("write your pallas kernel here")


# speedup vs baseline: 1.5088x; 1.5088x over previous
import functools

import jax
import jax.numpy as jnp
from jax.experimental import pallas as pl
from jax.experimental.pallas import tpu as pltpu

_F32 = jnp.float32
_BF16 = jnp.bfloat16
_N = 64
_NCP = 128
_VMEM_LIMIT = 56 * 1024 * 1024


def _relu_bf16(v):
    return jnp.maximum(v, 0.0).astype(_BF16)


def _shift_rows(y, s):
    if s == 0:
        return y
    z = jnp.zeros((abs(s), y.shape[1]), y.dtype)
    if s > 0:
        return jnp.concatenate([y[s:], z], axis=0)
    return jnp.concatenate([z, y[:s]], axis=0)


def _im2col_s1(y1, H, W):
    M, _ = y1.shape
    r = jax.lax.broadcasted_iota(jnp.int32, (M, 1), 0)
    rem = r % (H * W)
    yy = rem // W
    xx = rem % W
    cols = []
    for di in range(3):
        for dj in range(3):
            dy, dx = di - 1, dj - 1
            t = _shift_rows(y1, dy * W + dx)
            ok = ((yy + dy >= 0) & (yy + dy < H)
                  & (xx + dx >= 0) & (xx + dx < W))
            cols.append(jnp.where(ok, t, jnp.zeros_like(t)))
    return jnp.concatenate(cols, axis=1)


def _pool_mat(M):
    hw = M // _N
    rr = jax.lax.broadcasted_iota(jnp.int32, (_N, M), 0)
    cc = jax.lax.broadcasted_iota(jnp.int32, (_N, M), 1)
    return jnp.where(cc // hw == rr, 1.0 / hw, 0.0).astype(_BF16)


def _head_logsoftmax(y3, hw_ref, hb_ref):
    if y3.shape[0] == _N:
        pooled = y3
    else:
        pooled = jnp.dot(_pool_mat(y3.shape[0]), y3,
                         preferred_element_type=_F32).astype(_BF16)
    logits = jnp.dot(pooled, hw_ref[...], preferred_element_type=_F32) \
        + hb_ref[...]
    m = jnp.max(logits, axis=-1, keepdims=True)
    lse = jnp.log(jnp.sum(jnp.exp(logits - m), axis=-1, keepdims=True)) + m
    return logits - lse


def _bneck_math(x, c1w, c1b, c2w, c2b, c3w, c3b, H, W, idn_f32):
    y1 = _relu_bf16(jnp.dot(x, c1w, preferred_element_type=_F32) + c1b)
    if H == 1 and W == 1:
        a = y1
    else:
        a = _im2col_s1(y1, H, W)
    y2 = _relu_bf16(jnp.dot(a, c2w, preferred_element_type=_F32) + c2b)
    out = jnp.dot(y2, c3w, preferred_element_type=_F32) + c3b + idn_f32
    return _relu_bf16(out)


def _mm_bias_relu_body(a_ref, w_ref, b_ref, o_ref):
    o_ref[...] = _relu_bf16(
        jnp.dot(a_ref[...], w_ref[...], preferred_element_type=_F32)
        + b_ref[...])


def _block0_body(x9_ref, c1w, c1b, c2w, c2b, c3w, c3b, dw, db, xout_ref, *,
                 H, W):
    x = x9_ref[0]
    for t in range(1, 9):
        x = jnp.maximum(x, x9_ref[t])
    idn = (jnp.dot(x, dw[...], preferred_element_type=_F32)
           + db[...]).astype(_BF16)
    xout_ref[...] = _bneck_math(x, c1w[...], c1b[...], c2w[...], c2b[...],
                                c3w[...], c3b[...], H, W,
                                idn.astype(_F32))


def _stack_body(*refs, H, W, nb, with_exits, with_fc):
    i = 0
    x_ref = refs[i]; i += 1
    c1w, c1b, c2w, c2b, c3w, c3b = refs[i:i + 6]; i += 6
    if with_exits:
        ew, eb = refs[i:i + 2]; i += 2
    if with_fc:
        fcw, fcb = refs[i:i + 2]; i += 2
    if with_exits:
        ex_ref = refs[i]; i += 1
    if with_fc:
        fc_ref = refs[i]; i += 1
    xout_ref = refs[i]

    j = pl.program_id(0)

    @pl.when(j == 0)
    def _():
        xout_ref[...] = x_ref[...]

    x = xout_ref[...]
    y3 = _bneck_math(x, c1w[0], c1b[0], c2w[0], c2b[0], c3w[0], c3b[0],
                     H, W, x.astype(_F32))
    xout_ref[...] = y3
    if with_exits:
        ex_ref[0] = _head_logsoftmax(y3, ew.at[0], eb.at[0])
    if with_fc:
        @pl.when(j == nb - 1)
        def _():
            fc_ref[...] = _head_logsoftmax(y3, fcw, fcb)


def _tail_body(p_ref, xs_ref, c2w, c2b, c3w, c3b, dw, db, ew, eb,
               ex_ref, xout_ref):
    y2 = _relu_bf16(jnp.dot(p_ref[...], c2w[...],
                            preferred_element_type=_F32) + c2b[...])
    idn = (jnp.dot(xs_ref[...], dw[...], preferred_element_type=_F32)
           + db[...]).astype(_BF16)
    y3 = _relu_bf16(jnp.dot(y2, c3w[...], preferred_element_type=_F32)
                    + c3b[...] + idn.astype(_F32))
    xout_ref[...] = y3
    ex_ref[...] = _head_logsoftmax(y3, ew, eb)


def _cparams(*sem):
    return pltpu.CompilerParams(dimension_semantics=sem,
                                vmem_limit_bytes=_VMEM_LIMIT)


def _full(shape):
    return pl.BlockSpec(shape, lambda: tuple(0 for _ in shape))


def _mm_bias_relu(a, w, b, mt=None):
    M, _ = a.shape
    Nn = w.shape[1]
    if mt is None:
        mt = M
    return pl.pallas_call(
        _mm_bias_relu_body,
        out_shape=jax.ShapeDtypeStruct((M, Nn), _BF16),
        grid=(M // mt,),
        in_specs=[pl.BlockSpec((mt, a.shape[1]), lambda i: (i, 0)),
                  pl.BlockSpec(w.shape, lambda i: (0, 0)),
                  pl.BlockSpec(b.shape, lambda i: (0, 0))],
        out_specs=pl.BlockSpec((mt, Nn), lambda i: (i, 0)),
        compiler_params=_cparams("arbitrary"),
    )(a, w, b)


def _run_block0(x9, c1w, c1b, c2w, c2b, c3w, c3b, dw, db, H, W):
    M = x9.shape[1]
    cout = c3w.shape[1]
    return pl.pallas_call(
        functools.partial(_block0_body, H=H, W=W),
        out_shape=jax.ShapeDtypeStruct((M, cout), _BF16),
        in_specs=[_full(a.shape)
                  for a in (x9, c1w, c1b, c2w, c2b, c3w, c3b, dw, db)],
        out_specs=_full((M, cout)),
        compiler_params=_cparams(),
    )(x9, c1w, c1b, c2w, c2b, c3w, c3b, dw, db)


def _run_stack(x2d, blocks, exits, H, W, fc=None):
    nb = len(blocks)
    M, cin = x2d.shape
    p = blocks[0][0].shape[1]
    cout = 4 * p
    c1w = jnp.stack([bl[0] for bl in blocks])
    c1b = jnp.stack([bl[1] for bl in blocks])
    if H == 1 and W == 1:
        c2w = jnp.stack([bl[2][4 * p:5 * p] for bl in blocks])
    else:
        c2w = jnp.stack([bl[2] for bl in blocks])
    kk = c2w.shape[1]
    c2b = jnp.stack([bl[3] for bl in blocks])
    c3w = jnp.stack([bl[4] for bl in blocks])
    c3b = jnp.stack([bl[5] for bl in blocks])

    args = [x2d, c1w, c1b, c2w, c2b, c3w, c3b]
    in_specs = [
        pl.BlockSpec((M, cin), lambda j: (0, 0)),
        pl.BlockSpec((1, cin, p), lambda j: (j, 0, 0)),
        pl.BlockSpec((1, 1, p), lambda j: (j, 0, 0)),
        pl.BlockSpec((1, kk, p), lambda j: (j, 0, 0)),
        pl.BlockSpec((1, 1, p), lambda j: (j, 0, 0)),
        pl.BlockSpec((1, p, cout), lambda j: (j, 0, 0)),
        pl.BlockSpec((1, 1, cout), lambda j: (j, 0, 0)),
    ]
    out_shapes = []
    out_specs = []
    if exits is not None:
        args += [jnp.stack([e[0] for e in exits]),
                 jnp.stack([e[1] for e in exits])]
        in_specs += [pl.BlockSpec((1, cout, _NCP), lambda j: (j, 0, 0)),
                     pl.BlockSpec((1, 1, _NCP), lambda j: (j, 0, 0))]
        out_shapes.append(jax.ShapeDtypeStruct((nb, _N, _NCP), _F32))
        out_specs.append(pl.BlockSpec((1, _N, _NCP), lambda j: (j, 0, 0)))
    if fc is not None:
        args += [fc[0], fc[1]]
        in_specs += [pl.BlockSpec(fc[0].shape, lambda j: (0, 0)),
                     pl.BlockSpec(fc[1].shape, lambda j: (0, 0))]
        out_shapes.append(jax.ShapeDtypeStruct((_N, _NCP), _F32))
        out_specs.append(pl.BlockSpec((_N, _NCP), lambda j: (0, 0)))
    out_shapes.append(jax.ShapeDtypeStruct((M, cout), _BF16))
    out_specs.append(pl.BlockSpec((M, cout), lambda j: (0, 0)))

    body = functools.partial(_stack_body, H=H, W=W, nb=nb,
                             with_exits=exits is not None,
                             with_fc=fc is not None)
    res = pl.pallas_call(
        body,
        out_shape=tuple(out_shapes),
        grid=(nb,),
        in_specs=in_specs,
        out_specs=tuple(out_specs),
        compiler_params=_cparams("arbitrary"),
    )(*args)
    res = list(res)
    ex_out = res.pop(0) if exits is not None else None
    fc_out = res.pop(0) if fc is not None else None
    return res[0], ex_out, fc_out


def _run_tail(patches, xs, c2w, c2b, c3w, c3b, dw, db, ew, eb):
    M = patches.shape[0]
    cout = c3w.shape[1]
    outs = (jax.ShapeDtypeStruct((_N, _NCP), _F32),
            jax.ShapeDtypeStruct((M, cout), _BF16))
    ins = (patches, xs, c2w, c2b, c3w, c3b, dw, db, ew, eb)
    ex, xout = pl.pallas_call(
        _tail_body,
        out_shape=outs,
        in_specs=[_full(a.shape) for a in ins],
        out_specs=(_full((_N, _NCP)), _full((M, cout))),
        compiler_params=_cparams(),
    )(*ins)
    return xout, ex


def _im2col_xla(x4d, k, stride, pad):
    n, h, w, c = x4d.shape
    xp = jnp.pad(x4d, ((0, 0), (pad, pad), (pad, pad), (0, 0)))
    ho = (h + 2 * pad - k) // stride + 1
    wo = (w + 2 * pad - k) // stride + 1
    slabs = [xp[:, i:i + stride * ho:stride, j:j + stride * wo:stride, :]
             for i in range(k) for j in range(k)]
    patches = jnp.stack(slabs, axis=3)
    return patches.reshape(n * ho * wo, k * k * c), ho, wo


def _maxpool_slabs(x4d):
    n, h, w, c = x4d.shape
    lo = float(jnp.finfo(x4d.dtype).min)
    xp = jnp.pad(x4d, ((0, 0), (1, 1), (1, 1), (0, 0)), constant_values=lo)
    ho, wo = h // 2, w // 2
    slabs = [xp[:, i:i + 2 * ho:2, j:j + 2 * wo:2, :].reshape(n * ho * wo, c)
             for i in range(3) for j in range(3)]
    return jnp.stack(slabs, axis=0)


def _transition(x2d, shape4d, c1w, c1b, c2w, c2b, c3w, c3b, dw, db, ew, eb):
    n, h, w, cin = shape4d
    p = c1w.shape[1]
    y1 = _mm_bias_relu(x2d, c1w, c1b)
    patches, ho, wo = _im2col_xla(y1.reshape(n, h, w, p), 3, 2, 1)
    xs = x2d.reshape(n, h, w, cin)[:, ::2, ::2, :].reshape(n * ho * wo, cin)
    xout, ex = _run_tail(patches, xs, c2w, c2b, c3w, c3b, dw, db, ew, eb)
    return xout, ex, (n, ho, wo)


def kernel(x, sw, sb, *rest):
    ds_blocks = {0, 3, 11, 47}
    blocks = []
    i = 0
    for bi in range(50):
        c1w, c1b, c2w, c2b, c3w, c3b = rest[i:i + 6]
        i += 6
        if bi in ds_blocks:
            dwt, dbt = rest[i:i + 2]
            i += 2
        else:
            dwt = dbt = None
        blocks.append((c1w, c1b, c2w, c2b, c3w, c3b, dwt, dbt))
    exits = []
    for _ in range(47):
        exits.append((rest[i], rest[i + 1]))
        i += 2
    fcw, fcb = rest[i], rest[i + 1]

    xh = jnp.transpose(x, (0, 2, 3, 1)).astype(_BF16)
    patches, ho, wo = _im2col_xla(xh, 7, 2, 3)
    y = _mm_bias_relu(patches, sw, sb, mt=2048)

    x9 = _maxpool_slabs(y.reshape(_N, 16, 16, 64))
    b0 = blocks[0]
    xcur = _run_block0(x9, b0[0], b0[1], b0[2], b0[3], b0[4], b0[5],
                       b0[6], b0[7], 8, 8)

    xcur, _, _ = _run_stack(xcur, [b[:6] for b in blocks[1:3]], None, 8, 8)

    all_exits = []

    b3 = blocks[3]
    xcur, ex, _ = _transition(xcur, (_N, 8, 8, 256), b3[0], b3[1], b3[2],
                              b3[3], b3[4], b3[5], b3[6], b3[7],
                              exits[0][0], exits[0][1])
    all_exits.append(ex)
    xcur, exs, _ = _run_stack(xcur, [b[:6] for b in blocks[4:11]],
                              exits[1:8], 4, 4)
    all_exits.extend(exs[j] for j in range(7))

    b11 = blocks[11]
    xcur, ex, _ = _transition(xcur, (_N, 4, 4, 512), b11[0], b11[1], b11[2],
                              b11[3], b11[4], b11[5], b11[6], b11[7],
                              exits[8][0], exits[8][1])
    all_exits.append(ex)
    xcur, exs, _ = _run_stack(xcur, [b[:6] for b in blocks[12:47]],
                              exits[9:44], 2, 2)
    all_exits.extend(exs[j] for j in range(35))

    b47 = blocks[47]
    xcur, ex, _ = _transition(xcur, (_N, 2, 2, 1024), b47[0], b47[1], b47[2],
                              b47[3], b47[4], b47[5], b47[6], b47[7],
                              exits[44][0], exits[44][1])
    all_exits.append(ex)
    _, exs, fc_out = _run_stack(xcur, [b[:6] for b in blocks[48:50]],
                                exits[45:47], 1, 1, fc=(fcw, fcb))
    all_exits.extend(exs[j] for j in range(2))

    nc = 10
    return fc_out[:, :nc], [e[:, :nc] for e in all_exits]


# speedup vs baseline: 7.2962x; 4.8356x over previous
import functools

import jax
import jax.numpy as jnp
from jax.experimental import pallas as pl
from jax.experimental.pallas import tpu as pltpu

_F32 = jnp.float32
_BF16 = jnp.bfloat16
_N = 64
_NC = 10
_VMEM_LIMIT = 57 * 1024 * 1024

_PLAN = ([(8, 64, 64)] + [(8, 256, 64)] * 2
         + [(8, 256, 128)] + [(4, 512, 128)] * 7
         + [(4, 512, 256)] + [(2, 1024, 256)] * 35
         + [(2, 1024, 512)] + [(1, 2048, 512)] * 2)
_DS = {0, 3, 11, 47}
_STRIDE2 = {3, 11, 47}


def _relu_bf16(v):
    return jnp.maximum(v, 0.0).astype(_BF16)


def _shift_rows(y, s):
    if s == 0:
        return y
    z = jnp.zeros((abs(s), y.shape[1]), y.dtype)
    if s > 0:
        return jnp.concatenate([y[s:], z], axis=0)
    return jnp.concatenate([z, y[:s]], axis=0)


def _tap(y, H, W, dy, dx):
    M = y.shape[0]
    r = jax.lax.broadcasted_iota(jnp.int32, (M, 1), 0)
    rem = r % (H * W)
    yy = rem // W
    xx = rem % W
    ok = ((yy + dy >= 0) & (yy + dy < H) & (xx + dx >= 0) & (xx + dx < W))
    t = _shift_rows(y, dy * W + dx)
    return jnp.where(ok, t, jnp.zeros_like(t))


def _im2col_s1(y1, H, W):
    return jnp.concatenate(
        [_tap(y1, H, W, di - 1, dj - 1) for di in range(3) for dj in range(3)],
        axis=1)


def _sel_mat(Mi, H, W):
    Ho, Wo = H // 2, W // 2
    Mo = Mi // 4
    r = jax.lax.broadcasted_iota(jnp.int32, (Mo, 1), 0)
    n = r // (Ho * Wo)
    rem = r % (Ho * Wo)
    tgt = n * (H * W) + (rem // Wo) * 2 * W + (rem % Wo) * 2
    c = jax.lax.broadcasted_iota(jnp.int32, (Mo, Mi), 1)
    return (c == tgt).astype(_BF16)


def _pool_mat(M):
    hw = M // _N
    rr = jax.lax.broadcasted_iota(jnp.int32, (_N, M), 0)
    cc = jax.lax.broadcasted_iota(jnp.int32, (_N, M), 1)
    return jnp.where(cc // hw == rr, 1.0 / hw, 0.0).astype(_BF16)


def _head_out(y3, hw_ref, hb_ref, out_ref):
    if y3.shape[0] == _N:
        pooled = y3
    else:
        pooled = jnp.dot(_pool_mat(y3.shape[0]), y3,
                         preferred_element_type=_F32).astype(_BF16)
    logits = jnp.dot(pooled, hw_ref[...], preferred_element_type=_F32) \
        + hb_ref[...]
    m = jnp.max(logits, axis=-1, keepdims=True)
    lse = jnp.log(jnp.sum(jnp.exp(logits - m), axis=-1, keepdims=True)) + m
    out_ref[...] = (logits - lse)[:, :_NC]


def _stem(patches_ref, sw_ref, sb_ref):
    y = _relu_bf16(jnp.dot(patches_ref[...], sw_ref[...],
                           preferred_element_type=_F32) + sb_ref[...])
    ph = [y[q * 4096:(q + 1) * 4096] for q in range(4)]
    m = None
    for dyp in (-1, 0, 1):
        py, dy = (dyp & 1), (-1 if dyp < 0 else 0)
        for dxp in (-1, 0, 1):
            px, dx = (dxp & 1), (-1 if dxp < 0 else 0)
            t = _tap(ph[py * 2 + px], 8, 8, dy, dx)
            m = t if m is None else jnp.maximum(m, t)
    return m


def _bneck(x, H, W, stride, wr):
    y1 = _relu_bf16(jnp.dot(x, wr["c1w"][...],
                            preferred_element_type=_F32) + wr["c1b"][...])
    if H == 1 and W == 1:
        p = y1.shape[1]
        a, c2w = y1, wr["c2w"][4 * p:5 * p]
    else:
        a, c2w = _im2col_s1(y1, H, W), wr["c2w"][...]
    y2 = _relu_bf16(jnp.dot(a, c2w, preferred_element_type=_F32)
                    + wr["c2b"][...])
    idn = x
    if stride == 2:
        S = _sel_mat(x.shape[0], H, W)
        y2 = jnp.dot(S, y2, preferred_element_type=_F32).astype(_BF16)
        idn = jnp.dot(S, x, preferred_element_type=_F32).astype(_BF16)
    if "dw" in wr:
        idn = (jnp.dot(idn, wr["dw"][...], preferred_element_type=_F32)
               + wr["db"][...]).astype(_BF16)
    out = jnp.dot(y2, wr["c3w"][...], preferred_element_type=_F32) \
        + wr["c3b"][...] + idn.astype(_F32)
    return _relu_bf16(out)


def _net_body(*refs, cfg):
    i = 0
    if cfg["stem"]:
        patches_ref, sw_ref, sb_ref = refs[i:i + 3]
        i += 3
    else:
        x_ref = refs[i]
        i += 1
    blocks = []
    for bi in cfg["blocks"]:
        wr = {}
        for nm in ("c1w", "c1b", "c2w", "c2b", "c3w", "c3b"):
            wr[nm] = refs[i]
            i += 1
        if bi in _DS:
            wr["dw"], wr["db"] = refs[i:i + 2]
            i += 2
        if bi >= 3:
            wr["ew"], wr["eb"] = refs[i:i + 2]
            i += 2
        blocks.append(wr)
    if cfg["fc"]:
        fcw_ref, fcb_ref = refs[i:i + 2]
        i += 2
    outs = list(refs[i:])

    oi = 0
    if cfg["stem"]:
        x = _stem(patches_ref, sw_ref, sb_ref)
    else:
        x = x_ref[...]
    for bi, wr in zip(cfg["blocks"], blocks):
        H, ip, p = _PLAN[bi]
        x = _bneck(x, H, H, 2 if bi in _STRIDE2 else 1, wr)
        if bi >= 3:
            _head_out(x, wr["ew"], wr["eb"], outs[oi])
            oi += 1
    if cfg["fc"]:
        _head_out(x, fcw_ref, fcb_ref, outs[oi])
        oi += 1
    if cfg["xout"]:
        outs[oi][...] = x


def _run_segment(cfg, args, n_exits, xout_shape):
    out_shapes = [jax.ShapeDtypeStruct((_N, _NC), _F32)] * n_exits
    if cfg["fc"]:
        out_shapes.append(jax.ShapeDtypeStruct((_N, _NC), _F32))
    if cfg["xout"]:
        out_shapes.append(jax.ShapeDtypeStruct(xout_shape, _BF16))
    vmem = pl.BlockSpec(memory_space=pltpu.MemorySpace.VMEM)
    res = pl.pallas_call(
        functools.partial(_net_body, cfg=cfg),
        out_shape=tuple(out_shapes),
        in_specs=[vmem] * len(args),
        out_specs=tuple([vmem] * len(out_shapes)),
        compiler_params=pltpu.CompilerParams(vmem_limit_bytes=_VMEM_LIMIT),
    )(*args)
    return list(res)


def _stem_patches(x):
    xh = jnp.transpose(x, (0, 2, 3, 1)).astype(_BF16)
    xp = jnp.pad(xh, ((0, 0), (3, 3), (3, 3), (0, 0)))
    groups = []
    for py in range(2):
        for px in range(2):
            slabs = [xp[:, 2 * py + i:2 * py + i + 32:4,
                        2 * px + j:2 * px + j + 32:4, :]
                     for i in range(7) for j in range(7)]
            pt = jnp.stack(slabs, axis=3)
            groups.append(pt.reshape(4096, 147))
    return jnp.concatenate(groups, axis=0)


def kernel(x, sw, sb, *rest):
    w = {}
    i = 0
    for bi in range(50):
        for nm in ("c1w", "c1b", "c2w", "c2b", "c3w", "c3b"):
            w[f"b{bi}_{nm}"] = rest[i]
            i += 1
        if bi in _DS:
            w[f"b{bi}_dw"], w[f"b{bi}_db"] = rest[i:i + 2]
            i += 2
    for j in range(47):
        w[f"e{j}_w"], w[f"e{j}_b"] = rest[i:i + 2]
        i += 2
    fcw, fcb = rest[i], rest[i + 1]

    def block_args(bi):
        a = [w[f"b{bi}_{nm}"]
             for nm in ("c1w", "c1b", "c2w", "c2b", "c3w", "c3b")]
        if bi in _DS:
            a += [w[f"b{bi}_dw"], w[f"b{bi}_db"]]
        if bi >= 3:
            a += [w[f"e{bi - 3}_w"], w[f"e{bi - 3}_b"]]
        return a

    exits = []

    cfg = {"stem": True, "blocks": list(range(12)), "fc": False, "xout": True}
    args = [_stem_patches(x), sw, sb]
    for bi in cfg["blocks"]:
        args += block_args(bi)
    res = _run_segment(cfg, args, 9, (256, 1024))
    exits += res[:9]
    xcur = res[9]

    for lo, hi in ((12, 28), (28, 44)):
        cfg = {"stem": False, "blocks": list(range(lo, hi)), "fc": False,
               "xout": True}
        args = [xcur]
        for bi in cfg["blocks"]:
            args += block_args(bi)
        res = _run_segment(cfg, args, hi - lo, (256, 1024))
        exits += res[:hi - lo]
        xcur = res[hi - lo]

    cfg = {"stem": False, "blocks": list(range(44, 50)), "fc": True,
           "xout": False}
    args = [xcur]
    for bi in cfg["blocks"]:
        args += block_args(bi)
    args += [fcw, fcb]
    res = _run_segment(cfg, args, 6, None)
    exits += res[:6]
    final = res[6]

    return final, exits


# speedup vs baseline: 16.7674x; 2.2981x over previous
import functools

import jax
import jax.numpy as jnp
from jax.experimental import pallas as pl
from jax.experimental.pallas import tpu as pltpu

_F32 = jnp.float32
_BF16 = jnp.bfloat16
_N = 64
_NC = 10
_VMEM_LIMIT = 57 * 1024 * 1024

_PLAN = ([(8, 64, 64)] + [(8, 256, 64)] * 2
         + [(8, 256, 128)] + [(4, 512, 128)] * 7
         + [(4, 512, 256)] + [(2, 1024, 256)] * 35
         + [(2, 1024, 512)] + [(1, 2048, 512)] * 2)
_DS = {0, 3, 11, 47}
_STRIDE2 = {3, 11, 47}


def _relu_bf16(v):
    return jnp.maximum(v, 0.0).astype(_BF16)


def _shift_rows(y, s):
    if s == 0:
        return y
    z = jnp.zeros((abs(s), y.shape[1]), y.dtype)
    if s > 0:
        return jnp.concatenate([y[s:], z], axis=0)
    return jnp.concatenate([z, y[:s]], axis=0)


def _tap(y, H, W, dy, dx):
    M = y.shape[0]
    r = jax.lax.broadcasted_iota(jnp.int32, (M, 1), 0)
    rem = r % (H * W)
    yy = rem // W
    xx = rem % W
    ok = ((yy + dy >= 0) & (yy + dy < H) & (xx + dx >= 0) & (xx + dx < W))
    t = _shift_rows(y, dy * W + dx)
    return jnp.where(ok, t, jnp.zeros_like(t))


def _im2col_s1(y1, H, W):
    return jnp.concatenate(
        [_tap(y1, H, W, di - 1, dj - 1) for di in range(3) for dj in range(3)],
        axis=1)


def _sel_mat(Mi, H, W):
    Ho, Wo = H // 2, W // 2
    Mo = Mi // 4
    r = jax.lax.broadcasted_iota(jnp.int32, (Mo, 1), 0)
    n = r // (Ho * Wo)
    rem = r % (Ho * Wo)
    tgt = n * (H * W) + (rem // Wo) * 2 * W + (rem % Wo) * 2
    c = jax.lax.broadcasted_iota(jnp.int32, (Mo, Mi), 1)
    return (c == tgt).astype(_BF16)


def _pool_mat(M):
    hw = M // _N
    rr = jax.lax.broadcasted_iota(jnp.int32, (_N, M), 0)
    cc = jax.lax.broadcasted_iota(jnp.int32, (_N, M), 1)
    return jnp.where(cc // hw == rr, 1.0 / hw, 0.0).astype(_BF16)


def _head_out(y3, hw_ref, hb_ref, out_ref):
    if y3.shape[0] == _N:
        pooled = y3
    else:
        pooled = jnp.dot(_pool_mat(y3.shape[0]), y3,
                         preferred_element_type=_F32).astype(_BF16)
    logits = jnp.dot(pooled, hw_ref[...], preferred_element_type=_F32) \
        + hb_ref[...]
    m = jnp.max(logits, axis=-1, keepdims=True)
    lse = jnp.log(jnp.sum(jnp.exp(logits - m), axis=-1, keepdims=True)) + m
    out_ref[...] = (logits - lse)[:, :_NC]


def _stem(patches_ref, sw_ref, sb_ref):
    y = _relu_bf16(jnp.dot(patches_ref[...], sw_ref[...],
                           preferred_element_type=_F32) + sb_ref[...])
    ph = [y[q * 4096:(q + 1) * 4096] for q in range(4)]
    m = None
    for dyp in (-1, 0, 1):
        py, dy = (dyp & 1), (-1 if dyp < 0 else 0)
        for dxp in (-1, 0, 1):
            px, dx = (dxp & 1), (-1 if dxp < 0 else 0)
            t = _tap(ph[py * 2 + px], 8, 8, dy, dx)
            m = t if m is None else jnp.maximum(m, t)
    return m


def _bneck(x, H, W, stride, wr):
    y1 = _relu_bf16(jnp.dot(x, wr["c1w"][...],
                            preferred_element_type=_F32) + wr["c1b"][...])
    if H == 1 and W == 1:
        p = y1.shape[1]
        a, c2w = y1, wr["c2w"][4 * p:5 * p]
    else:
        a, c2w = _im2col_s1(y1, H, W), wr["c2w"][...]
    y2 = _relu_bf16(jnp.dot(a, c2w, preferred_element_type=_F32)
                    + wr["c2b"][...])
    idn = x
    if stride == 2:
        S = _sel_mat(x.shape[0], H, W)
        y2 = jnp.dot(S, y2, preferred_element_type=_F32).astype(_BF16)
        idn = jnp.dot(S, x, preferred_element_type=_F32).astype(_BF16)
    if "dw" in wr:
        idn = (jnp.dot(idn, wr["dw"][...], preferred_element_type=_F32)
               + wr["db"][...]).astype(_BF16)
    out = jnp.dot(y2, wr["c3w"][...], preferred_element_type=_F32) \
        + wr["c3b"][...] + idn.astype(_F32)
    return _relu_bf16(out)


def _net_body(*refs, cfg):
    i = 0
    if cfg["stem"]:
        patches_ref, sw_ref, sb_ref = refs[i:i + 3]
        i += 3
    else:
        x_ref = refs[i]
        i += 1
    blocks = []
    for bi in cfg["blocks"]:
        wr = {}
        for nm in ("c1w", "c1b", "c2w", "c2b", "c3w", "c3b"):
            wr[nm] = refs[i]
            i += 1
        if bi in _DS:
            wr["dw"], wr["db"] = refs[i:i + 2]
            i += 2
        if bi >= 3:
            wr["ew"], wr["eb"] = refs[i:i + 2]
            i += 2
        blocks.append(wr)
    if cfg["fc"]:
        fcw_ref, fcb_ref = refs[i:i + 2]
        i += 2
    outs = list(refs[i:])

    oi = 0
    if cfg["stem"]:
        x = _stem(patches_ref, sw_ref, sb_ref)
    else:
        x = x_ref[...]
    for bi, wr in zip(cfg["blocks"], blocks):
        H, ip, p = _PLAN[bi]
        x = _bneck(x, H, H, 2 if bi in _STRIDE2 else 1, wr)
        if bi >= 3:
            _head_out(x, wr["ew"], wr["eb"], outs[oi])
            oi += 1
    if cfg["fc"]:
        _head_out(x, fcw_ref, fcb_ref, outs[oi])
        oi += 1
    if cfg["xout"]:
        outs[oi][...] = x


def _run_segment(cfg, args, n_exits, xout_shape):
    out_shapes = [jax.ShapeDtypeStruct((_N, _NC), _F32)] * n_exits
    if cfg["fc"]:
        out_shapes.append(jax.ShapeDtypeStruct((_N, _NC), _F32))
    if cfg["xout"]:
        out_shapes.append(jax.ShapeDtypeStruct(xout_shape, _BF16))
    vmem = pl.BlockSpec(memory_space=pltpu.MemorySpace.VMEM)
    res = pl.pallas_call(
        functools.partial(_net_body, cfg=cfg),
        out_shape=tuple(out_shapes),
        in_specs=[vmem] * len(args),
        out_specs=tuple([vmem] * len(out_shapes)),
        compiler_params=pltpu.CompilerParams(vmem_limit_bytes=_VMEM_LIMIT),
    )(*args)
    return list(res)


def _stem_patches(x):
    xh = jnp.transpose(x, (0, 2, 3, 1)).astype(_BF16)
    pt = jax.lax.conv_general_dilated_patches(
        xh, (7, 7), (2, 2), ((3, 3), (3, 3)),
        dimension_numbers=("NHWC", "HWIO", "NHWC"))
    v = pt.reshape(_N, 8, 2, 8, 2, 147)
    v = jnp.transpose(v, (2, 4, 0, 1, 3, 5))
    return v.reshape(16384, 147)


def _sw_perm(sw):
    perm = [(i * 7 + j) * 3 + c
            for c in range(3) for i in range(7) for j in range(7)]
    return sw[jnp.array(perm)]


def kernel(x, sw, sb, *rest):
    w = {}
    i = 0
    for bi in range(50):
        for nm in ("c1w", "c1b", "c2w", "c2b", "c3w", "c3b"):
            w[f"b{bi}_{nm}"] = rest[i]
            i += 1
        if bi in _DS:
            w[f"b{bi}_dw"], w[f"b{bi}_db"] = rest[i:i + 2]
            i += 2
    for j in range(47):
        w[f"e{j}_w"], w[f"e{j}_b"] = rest[i:i + 2]
        i += 2
    fcw, fcb = rest[i], rest[i + 1]

    def block_args(bi):
        a = [w[f"b{bi}_{nm}"]
             for nm in ("c1w", "c1b", "c2w", "c2b", "c3w", "c3b")]
        if bi in _DS:
            a += [w[f"b{bi}_dw"], w[f"b{bi}_db"]]
        if bi >= 3:
            a += [w[f"e{bi - 3}_w"], w[f"e{bi - 3}_b"]]
        return a

    exits = []

    cfg = {"stem": True, "blocks": list(range(12)), "fc": False, "xout": True}
    args = [_stem_patches(x), _sw_perm(sw), sb]
    for bi in cfg["blocks"]:
        args += block_args(bi)
    res = _run_segment(cfg, args, 9, (256, 1024))
    exits += res[:9]
    xcur = res[9]

    for lo, hi in ((12, 28), (28, 44)):
        cfg = {"stem": False, "blocks": list(range(lo, hi)), "fc": False,
               "xout": True}
        args = [xcur]
        for bi in cfg["blocks"]:
            args += block_args(bi)
        res = _run_segment(cfg, args, hi - lo, (256, 1024))
        exits += res[:hi - lo]
        xcur = res[hi - lo]

    cfg = {"stem": False, "blocks": list(range(44, 50)), "fc": True,
           "xout": False}
    args = [xcur]
    for bi in cfg["blocks"]:
        args += block_args(bi)
    args += [fcw, fcb]
    res = _run_segment(cfg, args, 6, None)
    exits += res[:6]
    final = res[6]

    return final, exits
